# Initial kernel scaffold; baseline (speedup 1.0000x reference)
import jax
import jax.numpy as jnp
from jax.experimental import pallas as pl


def kernel(targets, predictions, mask):
    raise NotImplementedError("write your pallas kernel here")


# speedup vs baseline: 5.5684x; 5.5684x over previous
import functools

import jax
import jax.numpy as jnp
from jax.experimental import pallas as pl
from jax.experimental.pallas import tpu as pltpu

B = 8
H = 512
W = 512
NSHIFT = 29
SIZE = float(B * H * W)


def _loss_kernel(p_ref, m_ref, out_ref, desc_acc, d1_acc, m2_acc):
    b = pl.program_id(0)

    @pl.when(b == 0)
    def _init():
        desc_acc[...] = jnp.zeros((H, W), jnp.float32)
        d1_acc[...] = jnp.zeros((H, W), jnp.float32)
        m2_acc[...] = jnp.zeros((H, W), jnp.float32)

    p = jnp.round(p_ref[0])
    m = m_ref[0]
    a = p * m

    w_idx = jax.lax.broadcasted_iota(jnp.int32, (H, W), 1)
    h_idx = jax.lax.broadcasted_iota(jnp.int32, (H, W), 0)

    desc = jnp.zeros((H, W), jnp.float32)
    for shift in range(1, NSHIFT + 1):
        rolled = jnp.roll(a, -shift, axis=1)
        term = jnp.where(w_idx < (W - shift), rolled * m, 0.0)
        desc = desc + ((a - term) < 0.0).astype(jnp.float32)
    desc_acc[...] += desc

    s = jnp.where(a == 0.0, jnp.float32(jnp.inf), a)
    k = 2
    while k <= H:
        j = k // 2
        while j >= 1:
            up = jnp.roll(s, -j, axis=0)
            down = jnp.roll(s, j, axis=0)
            is_lower = (h_idx & j) == 0
            partner = jnp.where(is_lower, up, down)
            take_min = ((h_idx & k) == 0) == is_lower
            s = jnp.where(take_min, jnp.minimum(s, partner),
                          jnp.maximum(s, partner))
            j //= 2
        k *= 2

    n = jnp.sum((a != 0.0).astype(jnp.int32), axis=0, keepdims=True)
    i1 = jnp.maximum(n - 1, 0) // 2
    i2 = n // 2
    sel1 = jnp.sum(jnp.where(h_idx == i1, s, 0.0), axis=0, keepdims=True)
    sel2 = jnp.sum(jnp.where(h_idx == i2, s, 0.0), axis=0, keepdims=True)
    med = jnp.where(n == 0, 0.0, (sel1 + sel2) * 0.5)

    d1_acc[...] += (p != med).astype(jnp.float32)
    m2_acc[...] += m

    @pl.when(b == B - 1)
    def _finish():
        cell = desc_acc[...] * 20.0 + d1_acc[...] * m2_acc[...]
        total = jnp.sum(jnp.sum(cell, axis=1, keepdims=True),
                        axis=0, keepdims=True)
        out_ref[...] = total / SIZE


@jax.jit
def _loss(pred, mask):
    out = pl.pallas_call(
        _loss_kernel,
        grid=(B,),
        in_specs=[
            pl.BlockSpec((1, H, W), lambda b: (b, 0, 0)),
            pl.BlockSpec((1, H, W), lambda b: (b, 0, 0)),
        ],
        out_specs=pl.BlockSpec((1, 1), lambda b: (0, 0)),
        out_shape=jax.ShapeDtypeStruct((1, 1), jnp.float32),
        scratch_shapes=[
            pltpu.VMEM((H, W), jnp.float32),
            pltpu.VMEM((H, W), jnp.float32),
            pltpu.VMEM((H, W), jnp.float32),
        ],
    )(pred, mask)
    return out[0, 0]


def kernel(targets, predictions, mask):
    del targets
    pred = predictions.reshape(B, H, W)
    return _loss(pred, mask)


# speedup vs baseline: 8.9705x; 1.6110x over previous
import functools

import jax
import jax.numpy as jnp
from jax.experimental import pallas as pl
from jax.experimental.pallas import tpu as pltpu

B = 8
H = 512
W = 512
NSHIFT = 29
SIZE = float(B * H * W)


def _loss_kernel(p_ref, m_ref, out_ref, desc_acc, d1_acc, m2_acc):
    b = pl.program_id(0)

    @pl.when(b == 0)
    def _init():
        desc_acc[...] = jnp.zeros((H, W), jnp.float32)
        d1_acc[...] = jnp.zeros((H, W), jnp.float32)
        m2_acc[...] = jnp.zeros((H, W), jnp.float32)

    p = jnp.round(p_ref[0])
    m = m_ref[0]
    a = p * m

    w_idx = jax.lax.broadcasted_iota(jnp.int32, (H, W), 1)

    desc = jnp.zeros((H, W), jnp.float32)
    for shift in range(1, NSHIFT + 1):
        rolled = jnp.roll(a, -shift, axis=1)
        term = jnp.where(w_idx < (W - shift), rolled * m, 0.0)
        desc = desc + ((a - term) < 0.0).astype(jnp.float32)
    desc_acc[...] += desc

    s = jnp.where(a == 0.0, jnp.float32(jnp.inf), a)
    bits = jax.lax.bitcast_convert_type(s, jnp.int32)
    neg_key = jnp.bitwise_xor(jnp.bitwise_not(bits), jnp.int32(-2147483648))
    key = jnp.where(bits >= 0, bits, neg_key)

    n = jnp.sum((a != 0.0).astype(jnp.int32), axis=0, keepdims=True)
    k1 = jnp.maximum(n - 1, 0) // 2

    c_neg = jnp.sum((key < 0).astype(jnp.int32), axis=0, keepdims=True)
    is_neg = k1 < c_neg
    prefix = jnp.where(is_neg, jnp.int32(-1), jnp.int32(0))
    rank = jnp.where(is_neg, k1, k1 - c_neg)
    for t in range(30, -1, -1):
        high = key >> t
        two_p = prefix * 2
        cnt0 = jnp.sum((high == two_p).astype(jnp.int32), axis=0,
                       keepdims=True)
        pick0 = rank < cnt0
        prefix = jnp.where(pick0, two_p, two_p + 1)
        rank = jnp.where(pick0, rank, rank - cnt0)
    key1 = prefix

    c_eq = jnp.sum((key == key1).astype(jnp.int32), axis=0, keepdims=True)
    gt_key = jnp.where(key > key1, key, jnp.int32(2147483647))
    min_gt = jnp.min(gt_key, axis=0, keepdims=True)
    key2 = jnp.where(rank + 1 < c_eq, key1, min_gt)
    key2 = jnp.where((n % 2) == 1, key1, key2)

    def _to_f32(kk):
        bb = jnp.where(kk >= 0, kk,
                       jnp.bitwise_not(
                           jnp.bitwise_xor(kk, jnp.int32(-2147483648))))
        return jax.lax.bitcast_convert_type(bb, jnp.float32)

    med = jnp.where(n == 0, 0.0, (_to_f32(key1) + _to_f32(key2)) * 0.5)

    d1_acc[...] += (p != med).astype(jnp.float32)
    m2_acc[...] += m

    @pl.when(b == B - 1)
    def _finish():
        cell = desc_acc[...] * 20.0 + d1_acc[...] * m2_acc[...]
        total = jnp.sum(jnp.sum(cell, axis=1, keepdims=True),
                        axis=0, keepdims=True)
        out_ref[...] = total / SIZE


@jax.jit
def _loss(pred, mask):
    out = pl.pallas_call(
        _loss_kernel,
        grid=(B,),
        in_specs=[
            pl.BlockSpec((1, H, W), lambda b: (b, 0, 0)),
            pl.BlockSpec((1, H, W), lambda b: (b, 0, 0)),
        ],
        out_specs=pl.BlockSpec((1, 1), lambda b: (0, 0)),
        out_shape=jax.ShapeDtypeStruct((1, 1), jnp.float32),
        scratch_shapes=[
            pltpu.VMEM((H, W), jnp.float32),
            pltpu.VMEM((H, W), jnp.float32),
            pltpu.VMEM((H, W), jnp.float32),
        ],
    )(pred, mask)
    return out[0, 0]


def kernel(targets, predictions, mask):
    del targets
    pred = predictions.reshape(B, H, W)
    return _loss(pred, mask)


# speedup vs baseline: 9.4857x; 1.0574x over previous
import functools

import jax
import jax.numpy as jnp
from jax.experimental import pallas as pl
from jax.experimental.pallas import tpu as pltpu

B = 8
H = 512
W = 512
NSHIFT = 29
SIZE = float(B * H * W)


def _loss_kernel(p_ref, m_ref, out_ref, desc_acc, d1_acc, m2_acc):
    b = pl.program_id(0)

    @pl.when(b == 0)
    def _init():
        desc_acc[...] = jnp.zeros((H, W), jnp.float32)
        d1_acc[...] = jnp.zeros((H, W), jnp.float32)
        m2_acc[...] = jnp.zeros((H, W), jnp.float32)

    p = jnp.round(p_ref[0])
    m = m_ref[0]
    a = p * m

    apad = jnp.concatenate([a, jnp.zeros((H, 128), jnp.float32)], axis=1)
    cnt = jnp.zeros((H, W + 128), jnp.int32)
    for shift in range(1, NSHIFT + 1):
        rolled = jnp.roll(apad, -shift, axis=1)
        cnt = cnt + (apad < rolled).astype(jnp.int32)
    desc_acc[...] += cnt[:, :W].astype(jnp.float32) * m

    s = jnp.where(a == 0.0, jnp.float32(jnp.inf), a)
    bits = jax.lax.bitcast_convert_type(s, jnp.int32)
    neg_key = jnp.bitwise_xor(jnp.bitwise_not(bits), jnp.int32(-2147483648))
    key = jnp.where(bits >= 0, bits, neg_key)

    n = jnp.sum((a != 0.0).astype(jnp.int32), axis=0, keepdims=True)
    k1 = jnp.maximum(n - 1, 0) // 2

    c_neg = jnp.sum((key < 0).astype(jnp.int32), axis=0, keepdims=True)
    is_neg = k1 < c_neg
    prefix = jnp.where(is_neg, jnp.int32(-1), jnp.int32(0))
    rank = jnp.where(is_neg, k1, k1 - c_neg)
    for t in range(30, -1, -1):
        high = key >> t
        two_p = prefix * 2
        cnt0 = jnp.sum((high == two_p).astype(jnp.int32), axis=0,
                       keepdims=True)
        pick0 = rank < cnt0
        prefix = jnp.where(pick0, two_p, two_p + 1)
        rank = jnp.where(pick0, rank, rank - cnt0)
    key1 = prefix

    c_eq = jnp.sum((key == key1).astype(jnp.int32), axis=0, keepdims=True)
    gt_key = jnp.where(key > key1, key, jnp.int32(2147483647))
    min_gt = jnp.min(gt_key, axis=0, keepdims=True)
    key2 = jnp.where(rank + 1 < c_eq, key1, min_gt)
    key2 = jnp.where((n % 2) == 1, key1, key2)

    def _to_f32(kk):
        bb = jnp.where(kk >= 0, kk,
                       jnp.bitwise_not(
                           jnp.bitwise_xor(kk, jnp.int32(-2147483648))))
        return jax.lax.bitcast_convert_type(bb, jnp.float32)

    med = jnp.where(n == 0, 0.0, (_to_f32(key1) + _to_f32(key2)) * 0.5)

    d1_acc[...] += (p != med).astype(jnp.float32)
    m2_acc[...] += m

    @pl.when(b == B - 1)
    def _finish():
        cell = desc_acc[...] * 20.0 + d1_acc[...] * m2_acc[...]
        total = jnp.sum(jnp.sum(cell, axis=1, keepdims=True),
                        axis=0, keepdims=True)
        out_ref[...] = total / SIZE


@jax.jit
def _loss(pred, mask):
    out = pl.pallas_call(
        _loss_kernel,
        grid=(B,),
        in_specs=[
            pl.BlockSpec((1, H, W), lambda b: (b, 0, 0)),
            pl.BlockSpec((1, H, W), lambda b: (b, 0, 0)),
        ],
        out_specs=pl.BlockSpec((1, 1), lambda b: (0, 0)),
        out_shape=jax.ShapeDtypeStruct((1, 1), jnp.float32),
        scratch_shapes=[
            pltpu.VMEM((H, W), jnp.float32),
            pltpu.VMEM((H, W), jnp.float32),
            pltpu.VMEM((H, W), jnp.float32),
        ],
    )(pred, mask)
    return out[0, 0]


def kernel(targets, predictions, mask):
    del targets
    pred = predictions.reshape(B, H, W)
    return _loss(pred, mask)
